# Initial kernel scaffold; baseline (speedup 1.0000x reference)
import jax
import jax.numpy as jnp
from jax.experimental import pallas as pl


def kernel(x, edge_index, edge_attr, params):
    raise NotImplementedError("write your pallas kernel here")


# speedup vs baseline: 1.1095x; 1.1095x over previous
import functools

import jax
import jax.numpy as jnp
from jax import lax
from jax.experimental import pallas as pl
from jax.experimental.pallas import tpu as pltpu
from jax.experimental.pallas import tpu_sc as plsc

N = 10000
E = 160000
IN = 10
VDIM = 8
EIN = IN + VDIM
H = 16
OUT = 3
NUM_HIDDEN = 2

NC = 2
NS = 16
NW = NC * NS
CHUNK = 128
NCH = 40
EPT = NCH * CHUNK
EPAD = NW * EPT
NPAD = 10240
ETILE = 2048
PTILE = 2048
D1 = 32


def _make_sc_gather(D):
  mesh = plsc.VectorSubcoreMesh(core_axis_name="c", subcore_axis_name="s",
                                num_cores=NC, num_subcores=NS)

  @functools.partial(
      pl.kernel, mesh=mesh,
      compiler_params=pltpu.CompilerParams(use_tc_tiling_on_sc=False),
      out_type=jax.ShapeDtypeStruct((EPAD, D), jnp.float32),
      scratch_types=[
          pltpu.VMEM((NCH, CHUNK), jnp.int32),
          pltpu.VMEM((CHUNK, D), jnp.float32),
          pltpu.SemaphoreType.DMA,
      ],
  )
  def k(table_hbm, idx_hbm, out_hbm, idx_v, buf, sem):
    c = lax.axis_index("c")
    s = lax.axis_index("s")
    wid = c * NS + s
    pltpu.sync_copy(idx_hbm.at[wid], idx_v)
    base = wid * EPT

    def body(j, carry):
      pltpu.async_copy(table_hbm.at[idx_v.at[j]], buf, sem).wait()
      pltpu.sync_copy(buf, out_hbm.at[pl.ds(base + j * CHUNK, CHUNK)])
      return carry

    lax.fori_loop(0, NCH, body, 0)

  return k


def _make_sc_scatter():
  mesh = plsc.VectorSubcoreMesh(core_axis_name="c", subcore_axis_name="s",
                                num_cores=NC, num_subcores=NS)
  rows = NPAD // NS

  @functools.partial(
      pl.kernel, mesh=mesh,
      compiler_params=pltpu.CompilerParams(use_tc_tiling_on_sc=False),
      out_type=jax.ShapeDtypeStruct((NC * NPAD, H), jnp.float32),
      scratch_types=[
          pltpu.VMEM((NCH, CHUNK), jnp.int32),
          pltpu.VMEM((CHUNK, H), jnp.float32),
          pltpu.VMEM_SHARED((NPAD, H), jnp.float32),
      ],
  )
  def k(msg_hbm, idx_hbm, z_hbm, out_hbm, idx_v, buf, acc):
    c = lax.axis_index("c")
    s = lax.axis_index("s")
    wid = c * NS + s
    pltpu.sync_copy(z_hbm.at[pl.ds(s * rows, rows)],
                    acc.at[pl.ds(s * rows, rows)])
    pltpu.sync_copy(idx_hbm.at[wid], idx_v)
    plsc.subcore_barrier()
    base = wid * EPT

    def body(j, carry):
      pltpu.sync_copy(msg_hbm.at[pl.ds(base + j * CHUNK, CHUNK)], buf)
      pltpu.sync_copy(buf, acc.at[idx_v.at[j]], add=True)
      return carry

    lax.fori_loop(0, NCH, body, 0)
    plsc.subcore_barrier()
    pltpu.sync_copy(acc.at[pl.ds(s * rows, rows)],
                    out_hbm.at[pl.ds(c * NPAD + s * rows, rows)])

  return k


def _make_sc_count():
  mesh = plsc.VectorSubcoreMesh(core_axis_name="c", subcore_axis_name="s",
                                num_cores=NC, num_subcores=NS)
  rows = NPAD // NS

  @functools.partial(
      pl.kernel, mesh=mesh,
      compiler_params=pltpu.CompilerParams(use_tc_tiling_on_sc=False),
      out_type=jax.ShapeDtypeStruct((NC * NPAD, H), jnp.float32),
      scratch_types=[
          pltpu.VMEM((NCH, CHUNK), jnp.int32),
          pltpu.VMEM((CHUNK, H), jnp.float32),
          pltpu.VMEM_SHARED((NPAD, H), jnp.float32),
      ],
  )
  def k(ones_hbm, idx_hbm, z_hbm, out_hbm, idx_v, buf, acc):
    c = lax.axis_index("c")
    s = lax.axis_index("s")
    wid = c * NS + s
    pltpu.sync_copy(z_hbm.at[pl.ds(s * rows, rows)],
                    acc.at[pl.ds(s * rows, rows)])
    pltpu.sync_copy(idx_hbm.at[wid], idx_v)
    pltpu.sync_copy(ones_hbm, buf)
    plsc.subcore_barrier()

    def body(j, carry):
      pltpu.sync_copy(buf, acc.at[idx_v.at[j]], add=True)
      return carry

    lax.fori_loop(0, NCH, body, 0)
    plsc.subcore_barrier()
    pltpu.sync_copy(acc.at[pl.ds(s * rows, rows)],
                    out_hbm.at[pl.ds(c * NPAD + s * rows, rows)])

  return k


def _full(shape):
  return pl.BlockSpec(shape, lambda i: tuple(0 for _ in shape))


def _tiled(shape):
  return pl.BlockSpec(shape, lambda i: (i,) + tuple(0 for _ in shape[1:]))


def _prologue_call(x32, vw1, vb1, vw2, vb2, s2, s3):
  def body(x_ref, vw1_ref, vb1_ref, vw2_ref, vb2_ref, s2_ref, s3_ref,
           ex_ref, ve_ref):
    xb = x_ref[...]
    v = xb[:, 7:8]
    h1 = jnp.maximum(v * vw1_ref[...] + vb1_ref[...], 0.0)
    ve8 = jnp.dot(h1, vw2_ref[...], preferred_element_type=jnp.float32) \
        + vb2_ref[...]
    ex_ref[...] = xb + jnp.dot(ve8, s2_ref[...],
                               preferred_element_type=jnp.float32)
    ve_ref[...] = jnp.dot(ve8, s3_ref[...],
                          preferred_element_type=jnp.float32)

  return pl.pallas_call(
      body,
      grid=(NPAD // PTILE,),
      in_specs=[_tiled((PTILE, D1)), _full((1, VDIM)), _full((1, VDIM)),
                _full((VDIM, VDIM)), _full((1, VDIM)),
                _full((VDIM, D1)), _full((VDIM, H))],
      out_specs=[_tiled((PTILE, D1)), _tiled((PTILE, H))],
      out_shape=[jax.ShapeDtypeStruct((NPAD, D1), jnp.float32),
                 jax.ShapeDtypeStruct((NPAD, H), jnp.float32)],
  )(x32, vw1, vb1, vw2, vb2, s2, s3)


def _mid_call(xs, ea, t2r, bpad, w1, b1, din):
  def body(xs_ref, ea_ref, t_ref, b_ref, w1_ref, b1_ref, msg_ref):
    xsb = xs_ref[...]
    h = jnp.maximum(ea_ref[...] * w1_ref[...] + b1_ref[...], 0.0)
    y = jnp.dot(xsb, t_ref[...], preferred_element_type=jnp.float32)
    acc = jnp.dot(xsb, b_ref[...], preferred_element_type=jnp.float32)
    for kk in range(H):
      acc = acc + h[:, kk:kk + 1] * y[:, kk * H:(kk + 1) * H]
    msg_ref[...] = acc

  return pl.pallas_call(
      body,
      grid=(EPAD // ETILE,),
      in_specs=[_tiled((ETILE, din)), _tiled((ETILE, 1)),
                _full((din, H * H)), _full((din, H)),
                _full((1, H)), _full((1, H))],
      out_specs=_tiled((ETILE, H)),
      out_shape=jax.ShapeDtypeStruct((EPAD, H), jnp.float32),
  )(xs, ea, t2r, bpad, w1, b1)


def _epi_call(parts, cnts, xin, root, bias, alpha, beta, din, residual):
  def body(p_ref, c_ref, x_ref, r_ref, b_ref, a_ref, be_ref, o_ref):
    ssum = p_ref[0] + p_ref[1]
    cnt = (c_ref[0] + c_ref[1])[:, 0:1]
    mean = ssum / jnp.maximum(cnt, 1.0)
    xb = x_ref[...]
    o = mean + jnp.dot(xb, r_ref[...], preferred_element_type=jnp.float32) \
        + b_ref[...]
    o = jnp.maximum(o * a_ref[...] + be_ref[...], 0.0)
    if residual:
      o = o + xb
    o_ref[...] = o

  return pl.pallas_call(
      body,
      grid=(NPAD // PTILE,),
      in_specs=[pl.BlockSpec((2, PTILE, H), lambda i: (0, i, 0)),
                pl.BlockSpec((2, PTILE, H), lambda i: (0, i, 0)),
                _tiled((PTILE, din)), _full((din, H)), _full((1, H)),
                _full((1, H)), _full((1, H))],
      out_specs=_tiled((PTILE, H)),
      out_shape=jax.ShapeDtypeStruct((NPAD, H), jnp.float32),
  )(parts, cnts, xin, root, bias, alpha, beta)


def _final_call(parts, cnts, h, ve16, x32, oroot, obias, sw1a, sw1b, sb1,
                sw2, sb2):
  def body(p_ref, c_ref, h_ref, ve_ref, x_ref, or_ref, ob_ref, a_ref,
           bb_ref, sb1_ref, sw2_ref, sb2_ref, o_ref):
    ssum = p_ref[0] + p_ref[1]
    cnt = (c_ref[0] + c_ref[1])[:, 0:1]
    mean = ssum / jnp.maximum(cnt, 1.0)
    hb = h_ref[...]
    raw = mean + jnp.dot(hb, or_ref[...],
                         preferred_element_type=jnp.float32) + ob_ref[...]
    s1 = jnp.maximum(
        jnp.dot(hb, a_ref[...], preferred_element_type=jnp.float32)
        + jnp.dot(ve_ref[...], bb_ref[...],
                  preferred_element_type=jnp.float32)
        + sb1_ref[...], 0.0)
    s2 = jnp.dot(s1, sw2_ref[...], preferred_element_type=jnp.float32) \
        + sb2_ref[...]
    scale = 0.5 / (1.0 + jnp.exp(-s2[:, 0:1]))
    o_ref[...] = x_ref[:, 0:H] + raw * scale

  return pl.pallas_call(
      body,
      grid=(NPAD // PTILE,),
      in_specs=[pl.BlockSpec((2, PTILE, H), lambda i: (0, i, 0)),
                pl.BlockSpec((2, PTILE, H), lambda i: (0, i, 0)),
                _tiled((PTILE, H)), _tiled((PTILE, H)), _tiled((PTILE, D1)),
                _full((H, H)), _full((1, H)),
                _full((H, H)), _full((H, H)), _full((1, H)),
                _full((H, H)), _full((1, H))],
      out_specs=_tiled((PTILE, H)),
      out_shape=jax.ShapeDtypeStruct((NPAD, H), jnp.float32),
  )(parts, cnts, h, ve16, x32, oroot, obias, sw1a, sw1b, sb1, sw2, sb2)


def _pad_mat(m, r, c):
  return jnp.zeros((r, c), jnp.float32).at[:m.shape[0], :m.shape[1]].set(m)


def _prep_layer(w2, b2, root, din, dout, dpad):
  t = w2.reshape(H, din, dout).transpose(1, 0, 2)
  t2r = jnp.zeros((dpad, H, H), jnp.float32).at[:din, :, :dout].set(t)
  t2r = t2r.reshape(dpad, H * H)
  bp = _pad_mat(b2.reshape(din, dout), dpad, H)
  rp = _pad_mat(root, dpad, H)
  return t2r, bp, rp


@jax.jit
def kernel(x, edge_index, edge_attr, params):
  p = params
  src = edge_index[0].astype(jnp.int32)
  dst = edge_index[1].astype(jnp.int32)
  src_p = jnp.zeros((EPAD,), jnp.int32).at[:E].set(src).reshape(
      NW, NCH, CHUNK)
  dst_p = jnp.full((EPAD,), N, jnp.int32).at[:E].set(dst).reshape(
      NW, NCH, CHUNK)
  ea_p = jnp.zeros((EPAD, 1), jnp.float32).at[:E].set(edge_attr)

  x32 = jnp.zeros((NPAD, D1), jnp.float32).at[:N, :IN].set(x)
  zinit = jnp.zeros((NPAD, H), jnp.float32)
  ones_row = jnp.ones((CHUNK, H), jnp.float32)

  s2 = jnp.zeros((VDIM, D1), jnp.float32).at[
      jnp.arange(VDIM), IN + jnp.arange(VDIM)].set(1.0)
  s3 = jnp.zeros((VDIM, H), jnp.float32).at[
      jnp.arange(VDIM), jnp.arange(VDIM)].set(1.0)

  ex, ve16 = _prologue_call(
      x32, p['vw1'].reshape(1, VDIM), p['vb1'].reshape(1, VDIM),
      p['vw2'], p['vb2'].reshape(1, VDIM), s2, s3)

  gather32 = _make_sc_gather(D1)
  gather16 = _make_sc_gather(H)
  scatter = _make_sc_scatter()
  counter = _make_sc_count()

  cnts = counter(ones_row, dst_p, zinit).reshape(NC, NPAD, H)

  def conv(xin, din, gat, w1, b1, w2, b2, root, bias, dinp, dout):
    t2r, bp, rp = _prep_layer(w2, b2, root, dinp, dout, din)
    xs = gat(xin, src_p)
    msg = _mid_call(xs, ea_p, t2r, bp, w1.reshape(1, H), b1.reshape(1, H),
                    din)
    parts = scatter(msg, dst_p, zinit).reshape(NC, NPAD, H)
    return parts, rp

  inv = 1.0 / jnp.sqrt(1.0 + 1e-5)

  parts, rp = conv(ex, D1, gather32, p['iw1'], p['ib1'], p['iw2'], p['ib2'],
                   p['iroot'], p['ibias'], EIN, H)
  h = _epi_call(parts, cnts, ex, rp, p['ibias'].reshape(1, H),
                (p['ig'] * inv).reshape(1, H), p['ibeta'].reshape(1, H),
                D1, residual=False)

  for l in range(NUM_HIDDEN):
    parts, rp = conv(h, H, gather16, p[f'h{l}_w1'], p[f'h{l}_b1'],
                     p[f'h{l}_w2'], p[f'h{l}_b2'], p[f'h{l}_root'],
                     p[f'h{l}_bias'], H, H)
    h = _epi_call(parts, cnts, h, rp, p[f'h{l}_bias'].reshape(1, H),
                  (p[f'h{l}_g'] * inv).reshape(1, H),
                  p[f'h{l}_beta'].reshape(1, H), H, residual=True)

  t2r, bp, rp = _prep_layer(p['ow2'], p['ob2'], p['oroot'], H, OUT, H)
  xs = gather16(h, src_p)
  msg = _mid_call(xs, ea_p, t2r, bp, p['ow1'].reshape(1, H),
                  p['ob1'].reshape(1, H), H)
  parts = scatter(msg, dst_p, zinit).reshape(NC, NPAD, H)

  obias16 = jnp.zeros((1, H), jnp.float32).at[0, :OUT].set(p['obias'])
  sw1a = p['sw1'][:H]
  sw1b = _pad_mat(p['sw1'][H:], H, H)
  sw2p = _pad_mat(p['sw2'], H, H)
  sb2p = jnp.zeros((1, H), jnp.float32).at[0, 0].set(p['sb2'][0])

  out = _final_call(parts, cnts, h, ve16, x32, rp, obias16, sw1a, sw1b,
                    p['sb1'].reshape(1, H), sw2p, sb2p)
  return out[:N, :OUT]
